# Initial kernel scaffold; baseline (speedup 1.0000x reference)
import jax
import jax.numpy as jnp
from jax.experimental import pallas as pl


def kernel(delta_grad, embedding_matrix, src_embeds, pred_lm, rand_vals, src_tokens, attention_mask):
    raise NotImplementedError("write your pallas kernel here")


# speedup vs baseline: 23.7887x; 23.7887x over previous
import jax
import jax.numpy as jnp
from jax import lax
from jax.experimental import pallas as pl
from jax.experimental.pallas import tpu as pltpu
from jax.experimental.pallas import tpu_sc as plsc

B, L, D, K = 2, 128, 128, 100000
TOPK = 10
N_SPECIAL = 999
BL = B * L

NW = 32
RPW = BL // NW
NLANE = 16

KPAD = 102400
NBLK = 256
CHUNK = 16384
NFULL = K // CHUNK
TAILW = K - NFULL * CHUNK

NEG = float("-inf")


def _iota():
    return lax.iota(jnp.int32, NLANE)


def _splat_lane(vec, lane, fill):
    return jnp.max(jnp.where(_iota() == lane, vec, fill))


def _merge16(tv, ti, v, vi):
    sv, si = plsc.sort_key_val(v, vi, descending=True)
    rsv = lax.rev(sv, (0,))
    rsi = lax.rev(si, (0,))
    keep = tv >= rsv
    nv = jnp.where(keep, tv, rsv)
    ni = jnp.where(keep, ti, rsi)
    return plsc.sort_key_val(nv, ni, descending=True)


def _rsqrt(x):
    i = plsc.bitcast(x, jnp.int32)
    i = jnp.int32(0x5F3759DF) - (i >> 1)
    y = plsc.bitcast(i, jnp.float32)
    for _ in range(4):
        y = y * (jnp.float32(1.5) - jnp.float32(0.5) * x * y * y)
    return y


def _sc_body(pred_hbm, emb_hbm, dg_hbm, se_hbm, aux_hbm,
             filt_hbm, adv_hbm,
             row_buf, outbuf, dg_t, se_t, aux_vm, erows, sidx_r, sval_r,
             flip_vm, sem_g, sem_o, sem_s):
    w = lax.axis_index("s") * 2 + lax.axis_index("c")
    iota = _iota()
    neg16 = jnp.full((NLANE,), NEG, jnp.float32)
    zero16 = jnp.zeros((NLANE,), jnp.int32)

    def _fill_pad(t, _):
        row_buf[pl.ds(K + t * NLANE, NLANE)] = neg16
        return 0
    lax.fori_loop(0, (KPAD - K) // NLANE, _fill_pad, 0)

    def _fill_out(t, _):
        outbuf[pl.ds(t * NLANE, NLANE)] = neg16
        return 0
    lax.fori_loop(0, CHUNK // NLANE, _fill_out, 0)

    pltpu.sync_copy(dg_hbm.at[pl.ds(w * RPW * D, RPW * D)], dg_t)
    pltpu.sync_copy(se_hbm.at[pl.ds(w * RPW * D, RPW * D)], se_t)
    pltpu.sync_copy(aux_hbm.at[pl.ds(w * NLANE, NLANE)], aux_vm)
    va = aux_vm[...]

    def _row(j, flipacc):
        r = w * RPW + j
        rbase = r * K
        pltpu.sync_copy(pred_hbm.at[pl.ds(rbase, K)], row_buf.at[pl.ds(0, K)])

        def p1(it, accs):
            base = it * NBLK
            return tuple(
                jnp.maximum(accs[u], row_buf[pl.ds(base + u * NLANE, NLANE)])
                for u in range(16))
        accs = lax.fori_loop(0, KPAD // NBLK, p1, (neg16,) * 16)

        mval, mid = neg16, zero16
        for u in range(16):
            mval, mid = _merge16(mval, mid, accs[u], iota + u * NLANE)
        t10 = jnp.min(jnp.where(iota < TOPK, mval, jnp.float32(jnp.inf)))

        def p3_u(u, st):
            bmax = _splat_lane(mval, u, NEG)
            bid = _splat_lane(mid, u, 0)

            def scan_block(st2):
                def p3_t(t, st3):
                    ev, ei, ethr = st3
                    idxv = bid + NBLK * (t * NLANE) + NBLK * iota
                    v = plsc.load_gather(row_buf, [idxv])

                    def do_merge(st4):
                        ev4, ei4, _ = st4
                        ev5, ei5 = _merge16(ev4, ei4, v, idxv)
                        return (ev5, ei5,
                                jnp.broadcast_to(jnp.min(ev5), (NLANE,)))
                    return lax.cond(jnp.any(v > ethr), do_merge,
                                    lambda s: s, st3)
                return lax.fori_loop(0, KPAD // NBLK // NLANE, p3_t, st2)
            return lax.cond(bmax >= t10, scan_block, lambda s: s, st)

        eval_, eidx, _ = lax.fori_loop(0, 16, p3_u, (neg16, zero16, neg16))

        tok = _splat_lane(va, j, 0)
        att = _splat_lane(va, j + RPW, 0)
        eidx10 = eidx * att

        gh = []
        for jj in range(TOPK):
            tk = _splat_lane(eidx10, jj, 0)
            gh.append(pltpu.async_copy(
                emb_hbm.at[pl.ds(tk * D, D)],
                erows.at[pl.ds(jj * D, D)], sem_g))

        oh = []
        for c in range(NFULL):
            oh.append(pltpu.async_copy(
                outbuf.at[pl.ds(0, CHUNK)],
                filt_hbm.at[pl.ds(rbase + c * CHUNK, CHUNK)], sem_o))
        oh.append(pltpu.async_copy(
            outbuf.at[pl.ds(0, TAILW)],
            filt_hbm.at[pl.ds(rbase + NFULL * CHUNK, TAILW)], sem_o))

        def pdots(c, acc):
            pn, ps = acc
            d_c = dg_t[pl.ds(j * D + c * NLANE, NLANE)]
            s_c = se_t[pl.ds(j * D + c * NLANE, NLANE)]
            return (pn + d_c * s_c, ps + s_c * s_c)
        pn, ps = lax.fori_loop(0, D // NLANE, pdots,
                               (jnp.zeros((NLANE,), jnp.float32),) * 2)
        prev = jnp.sum(pn)
        sn = jnp.sum(ps)

        for h in gh:
            h.wait()

        def cdots(jj, acc):
            na, sa, ea = acc

            def inner(c, acc2):
                an, as_, ae = acc2
                d_c = dg_t[pl.ds(j * D + c * NLANE, NLANE)]
                s_c = se_t[pl.ds(j * D + c * NLANE, NLANE)]
                e_c = erows[pl.ds(jj * D + c * NLANE, NLANE)]
                return (an + d_c * e_c, as_ + s_c * e_c, ae + e_c * e_c)
            an, as_, ae = lax.fori_loop(
                0, D // NLANE, inner,
                (jnp.zeros((NLANE,), jnp.float32),) * 3)
            sel = iota == jj
            return (jnp.where(sel, jnp.sum(an), na),
                    jnp.where(sel, jnp.sum(as_), sa),
                    jnp.where(sel, jnp.sum(ae), ea))
        zf = jnp.zeros((NLANE,), jnp.float32)
        nacc, seacc, enacc = lax.fori_loop(0, TOPK, cdots, (zf, zf, zf))

        sq = jnp.maximum(enacc + sn - 2.0 * seacc, 0.0) + jnp.float32(1e-20)
        val = (nacc - prev) * _rsqrt(sq)
        valid = (eidx10 >= N_SPECIAL) & (eidx10 != tok) & (iota < TOPK)
        fval = jnp.where(valid, val, NEG)

        m = jnp.max(fval)
        flip = jnp.min(jnp.where(fval == m, eidx10, jnp.int32(K)))
        flip = jnp.where(m == NEG, jnp.int32(0), flip)
        flipacc = jnp.where(iota == j, flip, flipacc)

        i0 = _splat_lane(eidx10, 0, 0)
        v0 = _splat_lane(fval, 0, NEG)
        sidx_r[...] = rbase + jnp.where(iota < TOPK, eidx10, i0)
        sval_r[...] = jnp.where(iota < TOPK, fval, v0)
        for h in oh:
            h.wait()
        pltpu.async_copy(sval_r, filt_hbm.at[sidx_r], sem_s).wait()
        return flipacc

    flipacc = lax.fori_loop(0, RPW, _row, zero16)
    flip_vm[...] = flipacc
    pltpu.sync_copy(flip_vm, adv_hbm.at[pl.ds(w * NLANE, NLANE)])


@jax.jit
def kernel(delta_grad, embedding_matrix, src_embeds, pred_lm, rand_vals,
           src_tokens, attention_mask):
    pred = pred_lm.reshape(BL * K)
    emb = embedding_matrix.reshape(K * D)
    dg = delta_grad.reshape(BL * D)
    se = src_embeds.reshape(BL * D)
    tok = src_tokens.reshape(BL).astype(jnp.int32)
    att = attention_mask.reshape(BL).astype(jnp.int32)
    aux = jnp.concatenate(
        [tok.reshape(NW, RPW), att.reshape(NW, RPW)], axis=1).reshape(-1)

    mesh = plsc.VectorSubcoreMesh(core_axis_name="c", subcore_axis_name="s")
    filt, adv = pl.kernel(
        _sc_body,
        out_type=(
            jax.ShapeDtypeStruct((BL * K,), jnp.float32),
            jax.ShapeDtypeStruct((NW * NLANE,), jnp.int32),
        ),
        mesh=mesh,
        compiler_params=pltpu.CompilerParams(needs_layout_passes=False),
        scratch_types=[
            pltpu.VMEM((KPAD,), jnp.float32),
            pltpu.VMEM((CHUNK,), jnp.float32),
            pltpu.VMEM((RPW * D,), jnp.float32),
            pltpu.VMEM((RPW * D,), jnp.float32),
            pltpu.VMEM((NLANE,), jnp.int32),
            pltpu.VMEM((TOPK * D,), jnp.float32),
            pltpu.VMEM((NLANE,), jnp.int32),
            pltpu.VMEM((NLANE,), jnp.float32),
            pltpu.VMEM((NLANE,), jnp.int32),
            pltpu.SemaphoreType.DMA,
            pltpu.SemaphoreType.DMA,
            pltpu.SemaphoreType.DMA,
        ],
    )(pred, emb, dg, se, aux)

    adv_flip = adv.reshape(NW, NLANE)[:, :RPW].reshape(B, L)
    mask_idx = ((src_tokens >= N_SPECIAL) &
                (rand_vals > (1.0 - 0.3))).astype(src_tokens.dtype)
    adv_tokens = src_tokens * (1 - mask_idx) + adv_flip * mask_idx
    return adv_tokens, filt.reshape(B, L, K)


# speedup vs baseline: 24.6522x; 1.0363x over previous
import jax
import jax.numpy as jnp
from jax import lax
from jax.experimental import pallas as pl
from jax.experimental.pallas import tpu as pltpu
from jax.experimental.pallas import tpu_sc as plsc

B, L, D, K = 2, 128, 128, 100000
TOPK = 10
N_SPECIAL = 999
BL = B * L

NW = 32
RPW = BL // NW
NLANE = 16

KPAD = 102400
NBLK = 256
CHUNK = 16384
NFULL = K // CHUNK
TAILW = K - NFULL * CHUNK

NEG = float("-inf")


def _iota():
    return lax.iota(jnp.int32, NLANE)


def _splat_lane(vec, lane, fill):
    return jnp.max(jnp.where(_iota() == lane, vec, fill))


def _merge16(tv, ti, v, vi):
    sv, si = plsc.sort_key_val(v, vi, descending=True)
    rsv = lax.rev(sv, (0,))
    rsi = lax.rev(si, (0,))
    keep = tv >= rsv
    nv = jnp.where(keep, tv, rsv)
    ni = jnp.where(keep, ti, rsi)
    return plsc.sort_key_val(nv, ni, descending=True)


def _rsqrt(x):
    i = plsc.bitcast(x, jnp.int32)
    i = jnp.int32(0x5F3759DF) - (i >> 1)
    y = plsc.bitcast(i, jnp.float32)
    for _ in range(4):
        y = y * (jnp.float32(1.5) - jnp.float32(0.5) * x * y * y)
    return y


def _sc_body(pred_hbm, emb_hbm, dg_hbm, se_hbm, aux_hbm,
             filt_hbm, adv_hbm,
             row_buf, outbuf, dg_t, se_t, aux_vm, erows, rowvi, rowvv,
             flip_vm, sidx_refs, sval_refs, sem_g, sem_o, sem_s):
    w = lax.axis_index("s") * 2 + lax.axis_index("c")
    iota = _iota()
    neg16 = jnp.full((NLANE,), NEG, jnp.float32)
    zero16 = jnp.zeros((NLANE,), jnp.int32)

    def _fill_pad(t, _):
        row_buf[pl.ds(K + t * NLANE, NLANE)] = neg16
        return 0
    lax.fori_loop(0, (KPAD - K) // NLANE, _fill_pad, 0)

    def _fill_out(t, _):
        outbuf[pl.ds(t * NLANE, NLANE)] = neg16
        return 0
    lax.fori_loop(0, CHUNK // NLANE, _fill_out, 0)

    pltpu.sync_copy(dg_hbm.at[pl.ds(w * RPW * D, RPW * D)], dg_t)
    pltpu.sync_copy(se_hbm.at[pl.ds(w * RPW * D, RPW * D)], se_t)
    pltpu.sync_copy(aux_hbm.at[pl.ds(w * NLANE, NLANE)], aux_vm)
    va = aux_vm[...]

    def _row(j, flipacc):
        r = w * RPW + j
        rbase = r * K
        pltpu.sync_copy(pred_hbm.at[pl.ds(rbase, K)], row_buf.at[pl.ds(0, K)])

        for c in range(NFULL):
            pltpu.async_copy(
                outbuf.at[pl.ds(0, CHUNK)],
                filt_hbm.at[pl.ds(rbase + c * CHUNK, CHUNK)], sem_o)
        pltpu.async_copy(
            outbuf.at[pl.ds(0, TAILW)],
            filt_hbm.at[pl.ds(rbase + NFULL * CHUNK, TAILW)], sem_o)

        def p1(it, accs):
            base = it * NBLK
            return tuple(
                jnp.maximum(accs[u], row_buf[pl.ds(base + u * NLANE, NLANE)])
                for u in range(16))
        accs = lax.fori_loop(0, KPAD // NBLK, p1, (neg16,) * 16)

        mval, mid = neg16, zero16
        for u in range(16):
            mval, mid = _merge16(mval, mid, accs[u], iota + u * NLANE)
        t10 = jnp.min(jnp.where(iota < TOPK, mval, jnp.float32(jnp.inf)))

        def p3_u(u, st):
            bmax = _splat_lane(mval, u, NEG)
            bid = _splat_lane(mid, u, 0)

            def scan_block(st2):
                def p3_t(t, st3):
                    ev, ei, ethr = st3
                    idxv = bid + NBLK * (t * NLANE) + NBLK * iota
                    v = plsc.load_gather(row_buf, [idxv])

                    def do_merge(st4):
                        ev4, ei4, _ = st4
                        ev5, ei5 = _merge16(ev4, ei4, v, idxv)
                        return (ev5, ei5,
                                jnp.broadcast_to(jnp.min(ev5), (NLANE,)))
                    return lax.cond(jnp.any(v > ethr), do_merge,
                                    lambda s: s, st3)
                return lax.fori_loop(0, KPAD // NBLK // NLANE, p3_t, st2)
            return lax.cond(bmax >= t10, scan_block, lambda s: s, st)

        eval_, eidx, _ = lax.fori_loop(0, 16, p3_u, (neg16, zero16, neg16))

        tok = _splat_lane(va, j, 0)
        att = _splat_lane(va, j + RPW, 0)
        eidx10 = eidx * att

        gh = []
        for jj in range(TOPK):
            tk = _splat_lane(eidx10, jj, 0)
            gh.append(pltpu.async_copy(
                emb_hbm.at[pl.ds(tk * D, D)],
                erows.at[pl.ds(jj * D, D)], sem_g))

        def pdots(c, acc):
            pn, ps = acc
            d_c = dg_t[pl.ds(j * D + c * NLANE, NLANE)]
            s_c = se_t[pl.ds(j * D + c * NLANE, NLANE)]
            return (pn + d_c * s_c, ps + s_c * s_c)
        pn, ps = lax.fori_loop(0, D // NLANE, pdots,
                               (jnp.zeros((NLANE,), jnp.float32),) * 2)
        prev = jnp.sum(pn)
        sn = jnp.sum(ps)

        for h in gh:
            h.wait()

        def cdots(jj, acc):
            na, sa, ea = acc

            def inner(c, acc2):
                an, as_, ae = acc2
                d_c = dg_t[pl.ds(j * D + c * NLANE, NLANE)]
                s_c = se_t[pl.ds(j * D + c * NLANE, NLANE)]
                e_c = erows[pl.ds(jj * D + c * NLANE, NLANE)]
                return (an + d_c * e_c, as_ + s_c * e_c, ae + e_c * e_c)
            an, as_, ae = lax.fori_loop(
                0, D // NLANE, inner,
                (jnp.zeros((NLANE,), jnp.float32),) * 3)
            sel = iota == jj
            return (jnp.where(sel, jnp.sum(an), na),
                    jnp.where(sel, jnp.sum(as_), sa),
                    jnp.where(sel, jnp.sum(ae), ea))
        zf = jnp.zeros((NLANE,), jnp.float32)
        nacc, seacc, enacc = lax.fori_loop(0, TOPK, cdots, (zf, zf, zf))

        sq = jnp.maximum(enacc + sn - 2.0 * seacc, 0.0) + jnp.float32(1e-20)
        val = (nacc - prev) * _rsqrt(sq)
        valid = (eidx10 >= N_SPECIAL) & (eidx10 != tok) & (iota < TOPK)
        fval = jnp.where(valid, val, NEG)

        m = jnp.max(fval)
        flip = jnp.min(jnp.where(fval == m, eidx10, jnp.int32(K)))
        flip = jnp.where(m == NEG, jnp.int32(0), flip)
        flipacc = jnp.where(iota == j, flip, flipacc)

        i0 = _splat_lane(eidx10, 0, 0)
        v0 = _splat_lane(fval, 0, NEG)
        rowvi[pl.ds(j * NLANE, NLANE)] = jnp.where(iota < TOPK, eidx10, i0)
        rowvv[pl.ds(j * NLANE, NLANE)] = jnp.where(iota < TOPK, fval, v0)
        return flipacc

    flipacc = lax.fori_loop(0, RPW, _row, zero16)
    flip_vm[...] = flipacc
    pltpu.sync_copy(flip_vm, adv_hbm.at[pl.ds(w * NLANE, NLANE)])

    for j in range(RPW):
        rbase = (w * RPW + j) * K
        for c in range(NFULL):
            pltpu.make_async_copy(
                outbuf.at[pl.ds(0, CHUNK)],
                filt_hbm.at[pl.ds(rbase + c * CHUNK, CHUNK)], sem_o).wait()
        pltpu.make_async_copy(
            outbuf.at[pl.ds(0, TAILW)],
            filt_hbm.at[pl.ds(rbase + NFULL * CHUNK, TAILW)], sem_o).wait()

    sh = []
    for j in range(RPW):
        rbase = (w * RPW + j) * K
        sidx_refs[j][...] = rbase + rowvi[pl.ds(j * NLANE, NLANE)]
        sval_refs[j][...] = rowvv[pl.ds(j * NLANE, NLANE)]
        sh.append(pltpu.async_copy(
            sval_refs[j], filt_hbm.at[sidx_refs[j]], sem_s))
    for h in sh:
        h.wait()


@jax.jit
def kernel(delta_grad, embedding_matrix, src_embeds, pred_lm, rand_vals,
           src_tokens, attention_mask):
    pred = pred_lm.reshape(BL * K)
    emb = embedding_matrix.reshape(K * D)
    dg = delta_grad.reshape(BL * D)
    se = src_embeds.reshape(BL * D)
    tok = src_tokens.reshape(BL).astype(jnp.int32)
    att = attention_mask.reshape(BL).astype(jnp.int32)
    aux = jnp.concatenate(
        [tok.reshape(NW, RPW), att.reshape(NW, RPW)], axis=1).reshape(-1)

    mesh = plsc.VectorSubcoreMesh(core_axis_name="c", subcore_axis_name="s")
    filt, adv = pl.kernel(
        _sc_body,
        out_type=(
            jax.ShapeDtypeStruct((BL * K,), jnp.float32),
            jax.ShapeDtypeStruct((NW * NLANE,), jnp.int32),
        ),
        mesh=mesh,
        compiler_params=pltpu.CompilerParams(needs_layout_passes=False),
        scratch_types=[
            pltpu.VMEM((KPAD,), jnp.float32),
            pltpu.VMEM((CHUNK,), jnp.float32),
            pltpu.VMEM((RPW * D,), jnp.float32),
            pltpu.VMEM((RPW * D,), jnp.float32),
            pltpu.VMEM((NLANE,), jnp.int32),
            pltpu.VMEM((TOPK * D,), jnp.float32),
            pltpu.VMEM((RPW * NLANE,), jnp.int32),
            pltpu.VMEM((RPW * NLANE,), jnp.float32),
            pltpu.VMEM((NLANE,), jnp.int32),
            [pltpu.VMEM((NLANE,), jnp.int32) for _ in range(RPW)],
            [pltpu.VMEM((NLANE,), jnp.float32) for _ in range(RPW)],
            pltpu.SemaphoreType.DMA,
            pltpu.SemaphoreType.DMA,
            pltpu.SemaphoreType.DMA,
        ],
    )(pred, emb, dg, se, aux)

    adv_flip = adv.reshape(NW, NLANE)[:, :RPW].reshape(B, L)
    mask_idx = ((src_tokens >= N_SPECIAL) &
                (rand_vals > (1.0 - 0.3))).astype(src_tokens.dtype)
    adv_tokens = src_tokens * (1 - mask_idx) + adv_flip * mask_idx
    return adv_tokens, filt.reshape(B, L, K)


# speedup vs baseline: 25.4755x; 1.0334x over previous
import jax
import jax.numpy as jnp
from jax import lax
from jax.experimental import pallas as pl
from jax.experimental.pallas import tpu as pltpu
from jax.experimental.pallas import tpu_sc as plsc

B, L, D, K = 2, 128, 128, 100000
TOPK = 10
N_SPECIAL = 999
BL = B * L

NW = 32
RPW = BL // NW
NLANE = 16

KH = K // 2
KPADH = 53248
NBLK = 256
CHUNK = 16384
NFULL = K // CHUNK
TAILW = K - NFULL * CHUNK

NEG = float("-inf")


def _iota():
    return lax.iota(jnp.int32, NLANE)


def _splat_lane(vec, lane, fill):
    return jnp.max(jnp.where(_iota() == lane, vec, fill))


def _merge16(tv, ti, v, vi):
    sv, si = plsc.sort_key_val(v, vi, descending=True)
    rsv = lax.rev(sv, (0,))
    rsi = lax.rev(si, (0,))
    keep = tv >= rsv
    nv = jnp.where(keep, tv, rsv)
    ni = jnp.where(keep, ti, rsi)
    return plsc.sort_key_val(nv, ni, descending=True)


def _rsqrt(x):
    i = plsc.bitcast(x, jnp.int32)
    i = jnp.int32(0x5F3759DF) - (i >> 1)
    y = plsc.bitcast(i, jnp.float32)
    for _ in range(4):
        y = y * (jnp.float32(1.5) - jnp.float32(0.5) * x * y * y)
    return y


def _sc_body(pred_hbm, emb_hbm, dg_hbm, se_hbm, aux_hbm,
             filt_hbm, adv_hbm,
             buf_a, buf_b, outbuf, dg_t, se_t, aux_vm, erows, rowvi, rowvv,
             flip_vm, sidx_refs, sval_refs, sem_i, sem_g, sem_o, sem_s):
    w = lax.axis_index("s") * 2 + lax.axis_index("c")
    iota = _iota()
    neg16 = jnp.full((NLANE,), NEG, jnp.float32)
    zero16 = jnp.zeros((NLANE,), jnp.int32)

    def _fill_pad(t, _):
        buf_a[pl.ds(KH + t * NLANE, NLANE)] = neg16
        buf_b[pl.ds(KH + t * NLANE, NLANE)] = neg16
        return 0
    lax.fori_loop(0, (KPADH - KH) // NLANE, _fill_pad, 0)

    def _fill_out(t, _):
        outbuf[pl.ds(t * NLANE, NLANE)] = neg16
        return 0
    lax.fori_loop(0, CHUNK // NLANE, _fill_out, 0)

    pltpu.sync_copy(dg_hbm.at[pl.ds(w * RPW * D, RPW * D)], dg_t)
    pltpu.sync_copy(se_hbm.at[pl.ds(w * RPW * D, RPW * D)], se_t)
    pltpu.sync_copy(aux_hbm.at[pl.ds(w * NLANE, NLANE)], aux_vm)
    va = aux_vm[...]

    pltpu.async_copy(pred_hbm.at[pl.ds(w * RPW * K, KH)],
                     buf_a.at[pl.ds(0, KH)], sem_i)

    def _half_top16(buf, koff):
        def p1(it, accs):
            base = it * NBLK
            return tuple(
                jnp.maximum(accs[u], buf[pl.ds(base + u * NLANE, NLANE)])
                for u in range(16))
        accs = lax.fori_loop(0, KPADH // NBLK, p1, (neg16,) * 16)

        mval, mid = neg16, zero16
        for u in range(16):
            mval, mid = _merge16(mval, mid, accs[u], iota + u * NLANE)
        t10 = jnp.min(jnp.where(iota < TOPK, mval, jnp.float32(jnp.inf)))

        def p3_u(u, st):
            bmax = _splat_lane(mval, u, NEG)
            bid = _splat_lane(mid, u, 0)

            def scan_block(st2):
                def p3_t(t, st3):
                    ev, ei, ethr = st3
                    idxv = bid + NBLK * (t * NLANE) + NBLK * iota
                    v = plsc.load_gather(buf, [idxv])

                    def do_merge(st4):
                        ev4, ei4, _ = st4
                        ev5, ei5 = _merge16(ev4, ei4, v, idxv)
                        return (ev5, ei5,
                                jnp.broadcast_to(jnp.min(ev5), (NLANE,)))
                    return lax.cond(jnp.any(v > ethr), do_merge,
                                    lambda s: s, st3)
                return lax.fori_loop(0, KPADH // NBLK // NLANE, p3_t, st2)
            return lax.cond(bmax >= t10, scan_block, lambda s: s, st)

        ev, ei, _ = lax.fori_loop(0, 16, p3_u, (neg16, zero16, neg16))
        return ev, ei + koff

    def _wait_half(buf):
        pltpu.make_async_copy(
            pred_hbm.at[pl.ds(0, KH)], buf.at[pl.ds(0, KH)], sem_i).wait()

    def _row(j, flipacc):
        r = w * RPW + j
        rbase = r * K

        for c in range(NFULL):
            pltpu.async_copy(
                outbuf.at[pl.ds(0, CHUNK)],
                filt_hbm.at[pl.ds(rbase + c * CHUNK, CHUNK)], sem_o)
        pltpu.async_copy(
            outbuf.at[pl.ds(0, TAILW)],
            filt_hbm.at[pl.ds(rbase + NFULL * CHUNK, TAILW)], sem_o)

        _wait_half(buf_a)
        pltpu.async_copy(pred_hbm.at[pl.ds(rbase + KH, KH)],
                         buf_b.at[pl.ds(0, KH)], sem_i)
        av, ai = _half_top16(buf_a, 0)
        _wait_half(buf_b)

        @pl.when(j < RPW - 1)
        def _prefetch_next():
            pltpu.async_copy(pred_hbm.at[pl.ds(rbase + K, KH)],
                             buf_a.at[pl.ds(0, KH)], sem_i)

        bv, bi = _half_top16(buf_b, KH)
        eval_, eidx = _merge16(av, ai, bv, bi)

        tok = _splat_lane(va, j, 0)
        att = _splat_lane(va, j + RPW, 0)
        eidx10 = eidx * att

        gh = []
        for jj in range(TOPK):
            tk = _splat_lane(eidx10, jj, 0)
            gh.append(pltpu.async_copy(
                emb_hbm.at[pl.ds(tk * D, D)],
                erows.at[pl.ds(jj * D, D)], sem_g))

        def pdots(c, acc):
            pn, ps = acc
            d_c = dg_t[pl.ds(j * D + c * NLANE, NLANE)]
            s_c = se_t[pl.ds(j * D + c * NLANE, NLANE)]
            return (pn + d_c * s_c, ps + s_c * s_c)
        pn, ps = lax.fori_loop(0, D // NLANE, pdots,
                               (jnp.zeros((NLANE,), jnp.float32),) * 2)
        prev = jnp.sum(pn)
        sn = jnp.sum(ps)

        for h in gh:
            h.wait()

        def cdots(jj, acc):
            na, sa, ea = acc

            def inner(c, acc2):
                an, as_, ae = acc2
                d_c = dg_t[pl.ds(j * D + c * NLANE, NLANE)]
                s_c = se_t[pl.ds(j * D + c * NLANE, NLANE)]
                e_c = erows[pl.ds(jj * D + c * NLANE, NLANE)]
                return (an + d_c * e_c, as_ + s_c * e_c, ae + e_c * e_c)
            an, as_, ae = lax.fori_loop(
                0, D // NLANE, inner,
                (jnp.zeros((NLANE,), jnp.float32),) * 3)
            sel = iota == jj
            return (jnp.where(sel, jnp.sum(an), na),
                    jnp.where(sel, jnp.sum(as_), sa),
                    jnp.where(sel, jnp.sum(ae), ea))
        zf = jnp.zeros((NLANE,), jnp.float32)
        nacc, seacc, enacc = lax.fori_loop(0, TOPK, cdots, (zf, zf, zf))

        sq = jnp.maximum(enacc + sn - 2.0 * seacc, 0.0) + jnp.float32(1e-20)
        val = (nacc - prev) * _rsqrt(sq)
        valid = (eidx10 >= N_SPECIAL) & (eidx10 != tok) & (iota < TOPK)
        fval = jnp.where(valid, val, NEG)

        m = jnp.max(fval)
        flip = jnp.min(jnp.where(fval == m, eidx10, jnp.int32(K)))
        flip = jnp.where(m == NEG, jnp.int32(0), flip)
        flipacc = jnp.where(iota == j, flip, flipacc)

        i0 = _splat_lane(eidx10, 0, 0)
        v0 = _splat_lane(fval, 0, NEG)
        rowvi[pl.ds(j * NLANE, NLANE)] = jnp.where(iota < TOPK, eidx10, i0)
        rowvv[pl.ds(j * NLANE, NLANE)] = jnp.where(iota < TOPK, fval, v0)
        return flipacc

    flipacc = lax.fori_loop(0, RPW, _row, zero16)
    flip_vm[...] = flipacc
    pltpu.sync_copy(flip_vm, adv_hbm.at[pl.ds(w * NLANE, NLANE)])

    for j in range(RPW):
        rbase = (w * RPW + j) * K
        for c in range(NFULL):
            pltpu.make_async_copy(
                outbuf.at[pl.ds(0, CHUNK)],
                filt_hbm.at[pl.ds(rbase + c * CHUNK, CHUNK)], sem_o).wait()
        pltpu.make_async_copy(
            outbuf.at[pl.ds(0, TAILW)],
            filt_hbm.at[pl.ds(rbase + NFULL * CHUNK, TAILW)], sem_o).wait()

    sh = []
    for j in range(RPW):
        rbase = (w * RPW + j) * K
        sidx_refs[j][...] = rbase + rowvi[pl.ds(j * NLANE, NLANE)]
        sval_refs[j][...] = rowvv[pl.ds(j * NLANE, NLANE)]
        sh.append(pltpu.async_copy(
            sval_refs[j], filt_hbm.at[sidx_refs[j]], sem_s))
    for h in sh:
        h.wait()


@jax.jit
def kernel(delta_grad, embedding_matrix, src_embeds, pred_lm, rand_vals,
           src_tokens, attention_mask):
    pred = pred_lm.reshape(BL * K)
    emb = embedding_matrix.reshape(K * D)
    dg = delta_grad.reshape(BL * D)
    se = src_embeds.reshape(BL * D)
    tok = src_tokens.reshape(BL).astype(jnp.int32)
    att = attention_mask.reshape(BL).astype(jnp.int32)
    aux = jnp.concatenate(
        [tok.reshape(NW, RPW), att.reshape(NW, RPW)], axis=1).reshape(-1)

    mesh = plsc.VectorSubcoreMesh(core_axis_name="c", subcore_axis_name="s")
    filt, adv = pl.kernel(
        _sc_body,
        out_type=(
            jax.ShapeDtypeStruct((BL * K,), jnp.float32),
            jax.ShapeDtypeStruct((NW * NLANE,), jnp.int32),
        ),
        mesh=mesh,
        compiler_params=pltpu.CompilerParams(needs_layout_passes=False),
        scratch_types=[
            pltpu.VMEM((KPADH,), jnp.float32),
            pltpu.VMEM((KPADH,), jnp.float32),
            pltpu.VMEM((CHUNK,), jnp.float32),
            pltpu.VMEM((RPW * D,), jnp.float32),
            pltpu.VMEM((RPW * D,), jnp.float32),
            pltpu.VMEM((NLANE,), jnp.int32),
            pltpu.VMEM((TOPK * D,), jnp.float32),
            pltpu.VMEM((RPW * NLANE,), jnp.int32),
            pltpu.VMEM((RPW * NLANE,), jnp.float32),
            pltpu.VMEM((NLANE,), jnp.int32),
            [pltpu.VMEM((NLANE,), jnp.int32) for _ in range(RPW)],
            [pltpu.VMEM((NLANE,), jnp.float32) for _ in range(RPW)],
            pltpu.SemaphoreType.DMA,
            pltpu.SemaphoreType.DMA,
            pltpu.SemaphoreType.DMA,
            pltpu.SemaphoreType.DMA,
        ],
    )(pred, emb, dg, se, aux)

    adv_flip = adv.reshape(NW, NLANE)[:, :RPW].reshape(B, L)
    mask_idx = ((src_tokens >= N_SPECIAL) &
                (rand_vals > (1.0 - 0.3))).astype(src_tokens.dtype)
    adv_tokens = src_tokens * (1 - mask_idx) + adv_flip * mask_idx
    return adv_tokens, filt.reshape(B, L, K)
